# baseline (speedup 1.0000x reference)
import functools

import jax
import jax.numpy as jnp
from jax import lax
from jax.experimental import pallas as pl
from jax.experimental.pallas import tpu as pltpu
from jax.experimental.pallas import tpu_sc as plsc

G = 64
NPAD = 10112
NC, NS = 2, 16
NW = NC * NS
EK = 50
NB = 4
F = 128


def _sc_feature_scatter():
    E_W = 10000
    CH = E_W // EK
    ROWS_W = NPAD // NS

    mesh = plsc.VectorSubcoreMesh(core_axis_name="c", subcore_axis_name="s")
    scratch = [
        pltpu.VMEM((NB, EK), jnp.int32),
        pltpu.VMEM((NB, EK), jnp.int32),
        pltpu.VMEM_SHARED((NPAD, F), jnp.float32),
    ] + [pltpu.VMEM((EK, F), jnp.float32) for _ in range(NB)] \
      + [pltpu.SemaphoreType.DMA for _ in range(NB)]

    @functools.partial(
        pl.kernel,
        out_type=jax.ShapeDtypeStruct((NC, NPAD, F), jnp.float32),
        mesh=mesh,
        scratch_types=scratch,
    )
    def k(hp_hbm, row_hbm, col_hbm, zeros_hbm, out_hbm, rowv, colv, acc_sh,
          *bufs_sems):
        bufs = bufs_sems[:NB]
        sems = bufs_sems[NB:]
        cid = lax.axis_index("c")
        sid = lax.axis_index("s")
        wid = sid * NC + cid
        pltpu.sync_copy(zeros_hbm, acc_sh.at[pl.ds(sid * ROWS_W, ROWS_W)])
        plsc.subcore_barrier()

        def body(i, _):
            pltpu.sync_copy(row_hbm.at[wid, pl.ds(i * NB, NB)], rowv)
            pltpu.sync_copy(col_hbm.at[wid, pl.ds(i * NB, NB)], colv)
            descs = []
            for b in range(NB):
                descs.append(pltpu.async_copy(
                    hp_hbm.at[rowv.at[b]], bufs[b], sems[b]))
            for b in range(NB):
                descs[b].wait()
                pltpu.sync_copy(bufs[b], acc_sh.at[colv.at[b]], add=True)
            return ()

        lax.fori_loop(0, CH // NB, body, (), unroll=False)
        plsc.subcore_barrier()
        pltpu.sync_copy(acc_sh.at[pl.ds(sid * ROWS_W, ROWS_W)],
                        out_hbm.at[cid, pl.ds(sid * ROWS_W, ROWS_W)])

    return k


_sc_feature_scatter_built = None


def _edge_scatter_sc(hp, ei):
    global _sc_feature_scatter_built
    if _sc_feature_scatter_built is None:
        _sc_feature_scatter_built = _sc_feature_scatter()
    hp_pad = jnp.zeros((NPAD, F), jnp.float32).at[:hp.shape[0]].set(hp)
    row2 = ei[0].reshape(NW, 10000 // EK, EK)
    col2 = ei[1].reshape(NW, 10000 // EK, EK)
    zeros = jnp.zeros((NPAD // NS, F), jnp.float32)
    acc2 = _sc_feature_scatter_built(hp_pad, row2, col2, zeros)
    return acc2[0, :hp.shape[0]] + acc2[1, :hp.shape[0]]


def _radix_keep(s, bt, onehot):
    N = s.shape[0]
    b = jax.lax.bitcast_convert_type(s, jnp.int32)
    key = jnp.where(s >= 0, b ^ jnp.int32(-2147483648), ~b).astype(jnp.uint32)
    real = bt < G
    c = jnp.sum(onehot, axis=0).astype(jnp.int32)
    r = (c + 1) // 2
    alive = real
    keep_sure = jnp.zeros((N,), bool)

    def body(i, carry):
        alive, keep_sure, r = carry
        bpos = 31 - i
        bit1 = ((key >> bpos) & 1) == 1
        on = alive & bit1
        cnt1 = jnp.round(on.astype(jnp.float32) @ onehot).astype(jnp.int32)
        d = cnt1 >= r
        dn = (onehot @ d.astype(jnp.float32)) > 0.5
        keep_sure = keep_sure | (on & ~dn)
        r = jnp.where(d, r, r - cnt1)
        alive = alive & (bit1 == dn)
        return alive, keep_sure, r

    alive, keep_sure, r = jax.lax.fori_loop(0, 32, body, (alive, keep_sure, r))
    af = alive.astype(jnp.float32)
    ca = jnp.cumsum(af)
    a_g = af @ onehot
    excl = jnp.cumsum(a_g) - a_g
    tie_rank = ca - (onehot @ excl)
    rn = onehot @ r.astype(jnp.float32)
    keep = keep_sure | (alive & (tie_rank <= rn))
    return keep, c


def _gcn_pre(h, ei, W):
    N = h.shape[0]
    hw = h @ W
    deg = jnp.ones((N,), jnp.float32).at[ei[1]].add(1.0, mode='drop')
    dinv = jax.lax.rsqrt(deg)
    hp = dinv[:, None] * hw
    return hp, dinv


def _edge_scatter(hp, ei, N):
    hpx = jnp.concatenate([hp, jnp.zeros((1, hp.shape[1]), hp.dtype)], 0)
    rows = hpx[jnp.clip(ei[0], 0, N)]
    return jnp.zeros_like(hpx).at[ei[1]].add(rows, mode='drop')[:N]


def _gcn(h, ei, W, bvec):
    N = h.shape[0]
    hp, dinv = _gcn_pre(h, ei, W)
    if W.shape[1] == F:
        acc = _edge_scatter_sc(hp, ei)
    else:
        acc = _edge_scatter(hp, ei, N)
    return dinv[:, None] * (acc + hp) + bvec


def _head_kernel(r_ref, w1_ref, b1_ref, w2_ref, b2_ref, w3_ref, b3_ref, o_ref):
    r = r_ref[...]
    r = jnp.maximum(r @ w1_ref[...] + b1_ref[...], 0.0)
    r = jnp.maximum(r @ w2_ref[...] + b2_ref[...], 0.0)
    z = r @ w3_ref[...] + b3_ref[...]
    z = z - jnp.max(z, axis=-1, keepdims=True)
    o_ref[...] = z - jnp.log(jnp.sum(jnp.exp(z), axis=-1, keepdims=True))


def _head(r, l1W, l1b, l2W, l2b, l3W, l3b):
    C = l3W.shape[1]
    return pl.pallas_call(
        _head_kernel,
        out_shape=jax.ShapeDtypeStruct((r.shape[0], C), jnp.float32),
    )(r, l1W, l1b.reshape(1, -1), l2W, l2b.reshape(1, -1), l3W,
      l3b.reshape(1, -1))


def kernel(x, edge_index, batch, W1, b1, s1W, s1b, W2, b2, s2W, s2b,
           W3, b3, s3W, s3b, l1W, l1b, l2W, l2b, l3W, l3b):
    N = x.shape[0]
    bt = batch
    ei = edge_index
    iota_g = jnp.arange(G, dtype=jnp.int32)
    h = x
    xs = []
    for (W, bb, sW, sb) in ((W1, b1, s1W, s1b), (W2, b2, s2W, s2b),
                            (W3, b3, s3W, s3b)):
        h1 = jax.nn.relu(_gcn(h, ei, W, bb))
        s = _gcn(h1, ei, sW, sb)[:, 0]
        onehot = (bt[:, None] == iota_g[None, :]).astype(jnp.float32)
        keep, c = _radix_keep(s, bt, onehot)
        h = jnp.where(keep[:, None], h1 * jnp.tanh(s)[:, None], 0.0)
        bt = jnp.where(keep, bt, jnp.int32(G))
        kx = jnp.concatenate([keep, jnp.zeros((1,), bool)])
        good = kx[jnp.clip(ei[0], 0, N)] & kx[jnp.clip(ei[1], 0, N)]
        ei = jnp.where(good[None, :], ei, jnp.int32(N))
        keepoh = jnp.where(keep[:, None], onehot, 0.0)
        sm = keepoh.T @ h
        cnt = jnp.sum(keepoh, axis=0)
        btc = jnp.where(bt < G, bt, G)
        mx = jax.ops.segment_max(h, btc, num_segments=G + 1)[:G]
        xs.append(jnp.concatenate(
            [mx, sm / jnp.maximum(cnt, 1.0)[:, None]], axis=1))
    r = xs[0] + xs[1] + xs[2]
    return _head(r, l1W, l1b, l2W, l2b, l3W, l3b)


# speedup vs baseline: 2.2880x; 2.2880x over previous
import functools

import jax
import jax.numpy as jnp
from jax import lax
from jax.experimental import pallas as pl
from jax.experimental.pallas import tpu as pltpu
from jax.experimental.pallas import tpu_sc as plsc

G = 64
NPAD = 10112
NC, NS = 2, 16
NW = NC * NS
EK = 50
NB = 4
F = 128


def _sc_feature_scatter():
    E_W = 10000
    CH = E_W // EK
    ROWS_W = NPAD // NS

    mesh = plsc.VectorSubcoreMesh(core_axis_name="c", subcore_axis_name="s")
    scratch = [
        pltpu.VMEM((NB, EK), jnp.int32),
        pltpu.VMEM((NB, EK), jnp.int32),
        pltpu.VMEM_SHARED((NPAD, F), jnp.float32),
    ] + [pltpu.VMEM((EK, F), jnp.float32) for _ in range(NB)] \
      + [pltpu.SemaphoreType.DMA for _ in range(NB)]

    @functools.partial(
        pl.kernel,
        out_type=jax.ShapeDtypeStruct((NC, NPAD, F), jnp.float32),
        mesh=mesh,
        scratch_types=scratch,
    )
    def k(hp_hbm, row_hbm, col_hbm, zeros_hbm, out_hbm, rowv, colv, acc_sh,
          *bufs_sems):
        bufs = bufs_sems[:NB]
        sems = bufs_sems[NB:]
        cid = lax.axis_index("c")
        sid = lax.axis_index("s")
        wid = sid * NC + cid
        pltpu.sync_copy(zeros_hbm, acc_sh.at[pl.ds(sid * ROWS_W, ROWS_W)])
        plsc.subcore_barrier()

        def body(i, _):
            pltpu.sync_copy(row_hbm.at[wid, pl.ds(i * NB, NB)], rowv)
            pltpu.sync_copy(col_hbm.at[wid, pl.ds(i * NB, NB)], colv)
            descs = []
            for b in range(NB):
                descs.append(pltpu.async_copy(
                    hp_hbm.at[rowv.at[b]], bufs[b], sems[b]))
            for b in range(NB):
                descs[b].wait()
                pltpu.sync_copy(bufs[b], acc_sh.at[colv.at[b]], add=True)
            return ()

        lax.fori_loop(0, CH // NB, body, (), unroll=False)
        plsc.subcore_barrier()
        pltpu.sync_copy(acc_sh.at[pl.ds(sid * ROWS_W, ROWS_W)],
                        out_hbm.at[cid, pl.ds(sid * ROWS_W, ROWS_W)])

    return k


_sc_feature_scatter_built = None


def _edge_scatter_sc(hp, ei):
    global _sc_feature_scatter_built
    if _sc_feature_scatter_built is None:
        _sc_feature_scatter_built = _sc_feature_scatter()
    N = hp.shape[0]
    hp_pad = jnp.zeros((NPAD, F), jnp.float32).at[:N].set(hp)
    eidx = jnp.arange(ei.shape[1], dtype=jnp.int32)
    rowS = jnp.where(ei[0] < N, ei[0], N + eidx % (NPAD - N))
    colS = jnp.where(ei[1] < N, ei[1], eidx % N)
    row2 = rowS.reshape(NW, 10000 // EK, EK)
    col2 = colS.reshape(NW, 10000 // EK, EK)
    zeros = jnp.zeros((NPAD // NS, F), jnp.float32)
    acc2 = _sc_feature_scatter_built(hp_pad, row2, col2, zeros)
    return acc2[0, :hp.shape[0]] + acc2[1, :hp.shape[0]]


def _radix_keep(s, bt, onehot):
    N = s.shape[0]
    b = jax.lax.bitcast_convert_type(s, jnp.int32)
    key = jnp.where(s >= 0, b ^ jnp.int32(-2147483648), ~b).astype(jnp.uint32)
    real = bt < G
    c = jnp.sum(onehot, axis=0).astype(jnp.int32)
    r = (c + 1) // 2
    alive = real
    keep_sure = jnp.zeros((N,), bool)

    def body(i, carry):
        alive, keep_sure, r = carry
        bpos = 31 - i
        bit1 = ((key >> bpos) & 1) == 1
        on = alive & bit1
        cnt1 = jnp.round(on.astype(jnp.float32) @ onehot).astype(jnp.int32)
        d = cnt1 >= r
        dn = (onehot @ d.astype(jnp.float32)) > 0.5
        keep_sure = keep_sure | (on & ~dn)
        r = jnp.where(d, r, r - cnt1)
        alive = alive & (bit1 == dn)
        return alive, keep_sure, r

    alive, keep_sure, r = jax.lax.fori_loop(0, 32, body, (alive, keep_sure, r))
    af = alive.astype(jnp.float32)
    ca = jnp.cumsum(af)
    a_g = af @ onehot
    excl = jnp.cumsum(a_g) - a_g
    tie_rank = ca - (onehot @ excl)
    rn = onehot @ r.astype(jnp.float32)
    keep = keep_sure | (alive & (tie_rank <= rn))
    return keep, c


def _gcn_pre(h, ei, W):
    N = h.shape[0]
    hw = h @ W
    deg = jnp.ones((N,), jnp.float32).at[ei[1]].add(1.0, mode='drop')
    dinv = jax.lax.rsqrt(deg)
    hp = dinv[:, None] * hw
    return hp, dinv


def _edge_scatter(hp, ei, N):
    hpx = jnp.concatenate([hp, jnp.zeros((1, hp.shape[1]), hp.dtype)], 0)
    rows = hpx[jnp.clip(ei[0], 0, N)]
    return jnp.zeros_like(hpx).at[ei[1]].add(rows, mode='drop')[:N]


def _gcn(h, ei, W, bvec):
    N = h.shape[0]
    hp, dinv = _gcn_pre(h, ei, W)
    if W.shape[1] == F:
        acc = _edge_scatter_sc(hp, ei)
    else:
        acc = _edge_scatter(hp, ei, N)
    return dinv[:, None] * (acc + hp) + bvec


def _head_kernel(r_ref, w1_ref, b1_ref, w2_ref, b2_ref, w3_ref, b3_ref, o_ref):
    r = r_ref[...]
    r = jnp.maximum(r @ w1_ref[...] + b1_ref[...], 0.0)
    r = jnp.maximum(r @ w2_ref[...] + b2_ref[...], 0.0)
    z = r @ w3_ref[...] + b3_ref[...]
    z = z - jnp.max(z, axis=-1, keepdims=True)
    o_ref[...] = z - jnp.log(jnp.sum(jnp.exp(z), axis=-1, keepdims=True))


def _head(r, l1W, l1b, l2W, l2b, l3W, l3b):
    C = l3W.shape[1]
    return pl.pallas_call(
        _head_kernel,
        out_shape=jax.ShapeDtypeStruct((r.shape[0], C), jnp.float32),
    )(r, l1W, l1b.reshape(1, -1), l2W, l2b.reshape(1, -1), l3W,
      l3b.reshape(1, -1))


def kernel(x, edge_index, batch, W1, b1, s1W, s1b, W2, b2, s2W, s2b,
           W3, b3, s3W, s3b, l1W, l1b, l2W, l2b, l3W, l3b):
    N = x.shape[0]
    bt = batch
    ei = edge_index
    iota_g = jnp.arange(G, dtype=jnp.int32)
    h = x
    xs = []
    for (W, bb, sW, sb) in ((W1, b1, s1W, s1b), (W2, b2, s2W, s2b),
                            (W3, b3, s3W, s3b)):
        h1 = jax.nn.relu(_gcn(h, ei, W, bb))
        s = _gcn(h1, ei, sW, sb)[:, 0]
        onehot = (bt[:, None] == iota_g[None, :]).astype(jnp.float32)
        keep, c = _radix_keep(s, bt, onehot)
        h = jnp.where(keep[:, None], h1 * jnp.tanh(s)[:, None], 0.0)
        bt = jnp.where(keep, bt, jnp.int32(G))
        kx = jnp.concatenate([keep, jnp.zeros((1,), bool)])
        good = kx[jnp.clip(ei[0], 0, N)] & kx[jnp.clip(ei[1], 0, N)]
        ei = jnp.where(good[None, :], ei, jnp.int32(N))
        keepoh = jnp.where(keep[:, None], onehot, 0.0)
        sm = keepoh.T @ h
        cnt = jnp.sum(keepoh, axis=0)
        btc = jnp.where(bt < G, bt, G)
        mx = jax.ops.segment_max(h, btc, num_segments=G + 1)[:G]
        xs.append(jnp.concatenate(
            [mx, sm / jnp.maximum(cnt, 1.0)[:, None]], axis=1))
    r = xs[0] + xs[1] + xs[2]
    return _head(r, l1W, l1b, l2W, l2b, l3W, l3b)


# speedup vs baseline: 14.7738x; 6.4572x over previous
import functools

import jax
import jax.numpy as jnp
from jax import lax
from jax.experimental import pallas as pl
from jax.experimental.pallas import tpu as pltpu
from jax.experimental.pallas import tpu_sc as plsc

G = 64
NPAD = 10112
NC, NS = 2, 16
NW = NC * NS
EK = 50
NB = 4
F = 128


def _sc_feature_scatter():
    E_W = 10000
    CH = E_W // EK
    ROWS_W = NPAD // NS

    mesh = plsc.VectorSubcoreMesh(core_axis_name="c", subcore_axis_name="s")
    scratch = [
        pltpu.VMEM((NB, EK), jnp.int32),
        pltpu.VMEM((NB, EK), jnp.int32),
        pltpu.VMEM_SHARED((NPAD, F), jnp.float32),
    ] + [pltpu.VMEM((EK, F), jnp.float32) for _ in range(NB)] \
      + [pltpu.SemaphoreType.DMA for _ in range(NB)]

    @functools.partial(
        pl.kernel,
        out_type=jax.ShapeDtypeStruct((NC, NPAD, F), jnp.float32),
        mesh=mesh,
        scratch_types=scratch,
    )
    def k(hp_hbm, row_hbm, col_hbm, zeros_hbm, out_hbm, rowv, colv, acc_sh,
          *bufs_sems):
        bufs = bufs_sems[:NB]
        sems = bufs_sems[NB:]
        cid = lax.axis_index("c")
        sid = lax.axis_index("s")
        wid = sid * NC + cid
        pltpu.sync_copy(zeros_hbm, acc_sh.at[pl.ds(sid * ROWS_W, ROWS_W)])
        plsc.subcore_barrier()

        def body(i, _):
            pltpu.sync_copy(row_hbm.at[wid, pl.ds(i * NB, NB)], rowv)
            pltpu.sync_copy(col_hbm.at[wid, pl.ds(i * NB, NB)], colv)
            descs = []
            for b in range(NB):
                descs.append(pltpu.async_copy(
                    hp_hbm.at[rowv.at[b]], bufs[b], sems[b]))
            for b in range(NB):
                descs[b].wait()
                pltpu.sync_copy(bufs[b], acc_sh.at[colv.at[b]], add=True)
            return ()

        lax.fori_loop(0, CH // NB, body, (), unroll=False)
        plsc.subcore_barrier()
        pltpu.sync_copy(acc_sh.at[pl.ds(sid * ROWS_W, ROWS_W)],
                        out_hbm.at[cid, pl.ds(sid * ROWS_W, ROWS_W)])

    return k


_sc_feature_scatter_built = None


def _edge_scatter_sc(mat_pad, row2, col2):
    global _sc_feature_scatter_built
    if _sc_feature_scatter_built is None:
        _sc_feature_scatter_built = _sc_feature_scatter()
    zeros = jnp.zeros((NPAD // NS, F), jnp.float32)
    acc2 = _sc_feature_scatter_built(mat_pad, row2, col2, zeros)
    return acc2[0] + acc2[1]


def _pad_rows(v):
    return jnp.zeros((NPAD, F), jnp.float32).at[:v.shape[0], :v.shape[1]].set(v)


def _radix_keep(s, bt, onehot):
    N = s.shape[0]
    b = jax.lax.bitcast_convert_type(s, jnp.int32)
    key = jnp.where(s >= 0, b ^ jnp.int32(-2147483648), ~b).astype(jnp.uint32)
    real = bt < G
    c = jnp.sum(onehot, axis=0).astype(jnp.int32)
    r = (c + 1) // 2
    alive = real
    keep_sure = jnp.zeros((N,), bool)

    def body(i, carry):
        alive, keep_sure, r = carry
        bpos = 31 - i
        bit1 = ((key >> bpos) & 1) == 1
        on = alive & bit1
        cnt1 = jnp.round(on.astype(jnp.float32) @ onehot).astype(jnp.int32)
        d = cnt1 >= r
        dn = (onehot @ d.astype(jnp.float32)) > 0.5
        keep_sure = keep_sure | (on & ~dn)
        r = jnp.where(d, r, r - cnt1)
        alive = alive & (bit1 == dn)
        return alive, keep_sure, r

    alive, keep_sure, r = jax.lax.fori_loop(0, 32, body, (alive, keep_sure, r))
    af = alive.astype(jnp.float32)
    ca = jnp.cumsum(af)
    a_g = af @ onehot
    excl = jnp.cumsum(a_g) - a_g
    tie_rank = ca - (onehot @ excl)
    rn = onehot @ r.astype(jnp.float32)
    keep = keep_sure | (alive & (tie_rank <= rn))
    return keep, c


def _head_kernel(r_ref, w1_ref, b1_ref, w2_ref, b2_ref, w3_ref, b3_ref, o_ref):
    r = r_ref[...]
    r = jnp.maximum(r @ w1_ref[...] + b1_ref[...], 0.0)
    r = jnp.maximum(r @ w2_ref[...] + b2_ref[...], 0.0)
    z = r @ w3_ref[...] + b3_ref[...]
    z = z - jnp.max(z, axis=-1, keepdims=True)
    o_ref[...] = z - jnp.log(jnp.sum(jnp.exp(z), axis=-1, keepdims=True))


def _head(r, l1W, l1b, l2W, l2b, l3W, l3b):
    C = l3W.shape[1]
    return pl.pallas_call(
        _head_kernel,
        out_shape=jax.ShapeDtypeStruct((r.shape[0], C), jnp.float32),
    )(r, l1W, l1b.reshape(1, -1), l2W, l2b.reshape(1, -1), l3W,
      l3b.reshape(1, -1))


def kernel(x, edge_index, batch, W1, b1, s1W, s1b, W2, b2, s2W, s2b,
           W3, b3, s3W, s3b, l1W, l1b, l2W, l2b, l3W, l3b):
    N = x.shape[0]
    bt = batch
    row2 = edge_index[0].reshape(NW, 10000 // EK, EK)
    col2 = edge_index[1].reshape(NW, 10000 // EK, EK)
    iota_g = jnp.arange(G, dtype=jnp.int32)
    h = x
    kcum = jnp.ones((N,), jnp.float32)
    xs = []
    for (W, bb, sW, sb) in ((W1, b1, s1W, s1b), (W2, b2, s2W, s2b),
                            (W3, b3, s3W, s3b)):
        dacc = _edge_scatter_sc(_pad_rows(kcum[:, None]), row2, col2)
        deg = 1.0 + dacc[:N, 0]
        dinv = jax.lax.rsqrt(deg)
        hp = dinv[:, None] * (h @ W)
        acc = _edge_scatter_sc(_pad_rows(hp), row2, col2)
        h1 = jax.nn.relu(dinv[:, None] * (acc[:N] + hp) + bb)
        spp = dinv * (h1 @ sW)[:, 0]
        sacc = _edge_scatter_sc(_pad_rows(spp[:, None]), row2, col2)
        s = dinv * (sacc[:N, 0] + spp) + sb[0]
        onehot = (bt[:, None] == iota_g[None, :]).astype(jnp.float32)
        keep, _ = _radix_keep(s, bt, onehot)
        h = jnp.where(keep[:, None], h1 * jnp.tanh(s)[:, None], 0.0)
        bt = jnp.where(keep, bt, jnp.int32(G))
        kcum = keep.astype(jnp.float32)
        keepoh = jnp.where(keep[:, None], onehot, 0.0)
        sm = keepoh.T @ h
        cnt = jnp.sum(keepoh, axis=0)
        mx = jax.ops.segment_max(h, bt, num_segments=G + 1)[:G]
        xs.append(jnp.concatenate(
            [mx, sm / jnp.maximum(cnt, 1.0)[:, None]], axis=1))
    r = xs[0] + xs[1] + xs[2]
    return _head(r, l1W, l1b, l2W, l2b, l3W, l3b)


# speedup vs baseline: 14.7751x; 1.0001x over previous
import functools

import jax
import jax.numpy as jnp
from jax import lax
from jax.experimental import pallas as pl
from jax.experimental.pallas import tpu as pltpu
from jax.experimental.pallas import tpu_sc as plsc

G = 64
NPAD = 10112
NC, NS = 2, 16
NW = NC * NS
EK = 50
NB = 4
F = 128


def _sc_feature_scatter(width=F):
    E_W = 10000
    CH = E_W // EK
    ROWS_W = NPAD // NS

    mesh = plsc.VectorSubcoreMesh(core_axis_name="c", subcore_axis_name="s")
    scratch = [
        pltpu.VMEM((NB, EK), jnp.int32),
        pltpu.VMEM((NB, EK), jnp.int32),
        pltpu.VMEM_SHARED((NPAD, width), jnp.float32),
    ] + [pltpu.VMEM((EK, width), jnp.float32) for _ in range(NB)] \
      + [pltpu.SemaphoreType.DMA for _ in range(NB)]

    @functools.partial(
        pl.kernel,
        out_type=jax.ShapeDtypeStruct((NC, NPAD, width), jnp.float32),
        mesh=mesh,
        scratch_types=scratch,
    )
    def k(hp_hbm, row_hbm, col_hbm, zeros_hbm, out_hbm, rowv, colv, acc_sh,
          *bufs_sems):
        bufs = bufs_sems[:NB]
        sems = bufs_sems[NB:]
        cid = lax.axis_index("c")
        sid = lax.axis_index("s")
        wid = sid * NC + cid
        pltpu.sync_copy(zeros_hbm, acc_sh.at[pl.ds(sid * ROWS_W, ROWS_W)])
        plsc.subcore_barrier()

        def body(i, _):
            pltpu.sync_copy(row_hbm.at[wid, pl.ds(i * NB, NB)], rowv)
            pltpu.sync_copy(col_hbm.at[wid, pl.ds(i * NB, NB)], colv)
            descs = []
            for b in range(NB):
                descs.append(pltpu.async_copy(
                    hp_hbm.at[rowv.at[b]], bufs[b], sems[b]))
            for b in range(NB):
                descs[b].wait()
                pltpu.sync_copy(bufs[b], acc_sh.at[colv.at[b]], add=True)
            return ()

        lax.fori_loop(0, CH // NB, body, (), unroll=False)
        plsc.subcore_barrier()
        pltpu.sync_copy(acc_sh.at[pl.ds(sid * ROWS_W, ROWS_W)],
                        out_hbm.at[cid, pl.ds(sid * ROWS_W, ROWS_W)])

    return k


_sc_scatter_built = {}


def _edge_scatter_sc(mat_pad, row2, col2):
    w = mat_pad.shape[1]
    if w not in _sc_scatter_built:
        _sc_scatter_built[w] = _sc_feature_scatter(w)
    zeros = jnp.zeros((NPAD // NS, w), jnp.float32)
    acc2 = _sc_scatter_built[w](mat_pad, row2, col2, zeros)
    return acc2[0] + acc2[1]


def _pad_rows(v, width=F):
    return jnp.zeros((NPAD, width),
                     jnp.float32).at[:v.shape[0], :v.shape[1]].set(v)


def _radix_keep(s, bt, onehot):
    N = s.shape[0]
    b = jax.lax.bitcast_convert_type(s, jnp.int32)
    key = jnp.where(s >= 0, b ^ jnp.int32(-2147483648), ~b).astype(jnp.uint32)
    real = bt < G
    c = jnp.sum(onehot, axis=0).astype(jnp.int32)
    r = (c + 1) // 2
    alive = real
    keep_sure = jnp.zeros((N,), bool)

    def body(i, carry):
        alive, keep_sure, r = carry
        bpos = 31 - i
        bit1 = ((key >> bpos) & 1) == 1
        on = alive & bit1
        cnt1 = jnp.round(on.astype(jnp.float32) @ onehot).astype(jnp.int32)
        d = cnt1 >= r
        dn = (onehot @ d.astype(jnp.float32)) > 0.5
        keep_sure = keep_sure | (on & ~dn)
        r = jnp.where(d, r, r - cnt1)
        alive = alive & (bit1 == dn)
        return alive, keep_sure, r

    alive, keep_sure, r = jax.lax.fori_loop(0, 32, body, (alive, keep_sure, r))
    af = alive.astype(jnp.float32)
    ca = jnp.cumsum(af)
    a_g = af @ onehot
    excl = jnp.cumsum(a_g) - a_g
    tie_rank = ca - (onehot @ excl)
    rn = onehot @ r.astype(jnp.float32)
    keep = keep_sure | (alive & (tie_rank <= rn))
    return keep, c


def _head_kernel(r_ref, w1_ref, b1_ref, w2_ref, b2_ref, w3_ref, b3_ref, o_ref):
    r = r_ref[...]
    r = jnp.maximum(r @ w1_ref[...] + b1_ref[...], 0.0)
    r = jnp.maximum(r @ w2_ref[...] + b2_ref[...], 0.0)
    z = r @ w3_ref[...] + b3_ref[...]
    z = z - jnp.max(z, axis=-1, keepdims=True)
    o_ref[...] = z - jnp.log(jnp.sum(jnp.exp(z), axis=-1, keepdims=True))


def _head(r, l1W, l1b, l2W, l2b, l3W, l3b):
    C = l3W.shape[1]
    return pl.pallas_call(
        _head_kernel,
        out_shape=jax.ShapeDtypeStruct((r.shape[0], C), jnp.float32),
    )(r, l1W, l1b.reshape(1, -1), l2W, l2b.reshape(1, -1), l3W,
      l3b.reshape(1, -1))


def kernel(x, edge_index, batch, W1, b1, s1W, s1b, W2, b2, s2W, s2b,
           W3, b3, s3W, s3b, l1W, l1b, l2W, l2b, l3W, l3b):
    N = x.shape[0]
    bt = batch
    row2 = edge_index[0].reshape(NW, 10000 // EK, EK)
    col2 = edge_index[1].reshape(NW, 10000 // EK, EK)
    iota_g = jnp.arange(G, dtype=jnp.int32)
    h = x
    kcum = jnp.ones((N,), jnp.float32)
    xs = []
    for (W, bb, sW, sb) in ((W1, b1, s1W, s1b), (W2, b2, s2W, s2b),
                            (W3, b3, s3W, s3b)):
        dacc = _edge_scatter_sc(_pad_rows(kcum[:, None]), row2, col2)
        deg = 1.0 + dacc[:N, 0]
        dinv = jax.lax.rsqrt(deg)
        hp = dinv[:, None] * (h @ W)
        acc = _edge_scatter_sc(_pad_rows(hp), row2, col2)
        h1 = jax.nn.relu(dinv[:, None] * (acc[:N] + hp) + bb)
        spp = dinv * (h1 @ sW)[:, 0]
        sacc = _edge_scatter_sc(_pad_rows(spp[:, None]), row2, col2)
        s = dinv * (sacc[:N, 0] + spp) + sb[0]
        onehot = (bt[:, None] == iota_g[None, :]).astype(jnp.float32)
        keep, _ = _radix_keep(s, bt, onehot)
        h = jnp.where(keep[:, None], h1 * jnp.tanh(s)[:, None], 0.0)
        bt = jnp.where(keep, bt, jnp.int32(G))
        kcum = keep.astype(jnp.float32)
        keepoh = jnp.where(keep[:, None], onehot, 0.0)
        sm = keepoh.T @ h
        cnt = jnp.sum(keepoh, axis=0)
        mx = jax.ops.segment_max(h, bt, num_segments=G + 1)[:G]
        xs.append(jnp.concatenate(
            [mx, sm / jnp.maximum(cnt, 1.0)[:, None]], axis=1))
    r = xs[0] + xs[1] + xs[2]
    return _head(r, l1W, l1b, l2W, l2b, l3W, l3b)
